# baseline (speedup 1.0000x reference)
import jax
import jax.numpy as jnp
from jax import lax
from jax.experimental import pallas as pl
from jax.experimental.pallas import tpu as pltpu
from jax.experimental.pallas import tpu_sc as plsc

N = 10000
E = 320000
D = 128
H = 64
C = 10
G = 128

NC = 2
NS = 16
NW = NC * NS

K = 128
NBLK = 80
K2 = 128
NBLK2 = 80
NBLK_A = 80
NBLK_B = 80
NBLK_MAX = max(NBLK_A, NBLK_B)
EPT = K * NBLK
E_PAD = EPT * NW
N_PAD = 10240
STRIPE = N_PAD // NS
HALF = STRIPE // 2
HP = 64

RB = 1280
TGRID = N_PAD // RB

def _mesh():
    return plsc.VectorSubcoreMesh(core_axis_name="c", subcore_axis_name="s",
                                  num_cores=NC, num_subcores=NS)


DEG_R = N_PAD // 128


def _deg_body(dst2, zeros_n, iota_r, out_hbm, di_flat, table_v, idx_r, deg_sh):
    c = lax.axis_index("c")
    s = lax.axis_index("s")
    eid = c * NS + s
    pltpu.sync_copy(zeros_n, table_v)

    @pl.when(s == 0)
    def _zero_shared():
        pltpu.sync_copy(table_v, deg_sh)

    pltpu.sync_copy(dst2.at[eid], di_flat)
    pltpu.sync_copy(iota_r, idx_r)
    plsc.subcore_barrier()

    ones16 = jnp.ones((16,), jnp.float32)

    def body(i, carry):
        idx = di_flat[pl.ds(i * 16, 16)]
        plsc.addupdate_scatter(table_v, [idx >> 7, idx & 127], ones16)
        return carry

    lax.fori_loop(0, EPT // 16, body, 0)
    pltpu.sync_copy(table_v, deg_sh.at[idx_r], add=True)
    plsc.subcore_barrier()
    @pl.when(s < DEG_R // 8)
    def _copy_out():
        pltpu.sync_copy(deg_sh.at[pl.ds(s * 8, 8)],
                        table_v.at[pl.ds(0, 8)])
        pltpu.sync_copy(table_v.at[pl.ds(0, 8)],
                        out_hbm.at[c].at[pl.ds(s * 8, 8)])


def _sc_degree(dst2, zeros_n, iota_r):
    kfn = pl.kernel(
        _deg_body,
        out_type=jax.ShapeDtypeStruct((NC, DEG_R, 128), jnp.float32),
        mesh=_mesh(),
        compiler_params=pltpu.CompilerParams(needs_layout_passes=False),
        scratch_types=[
            pltpu.VMEM((EPT,), jnp.int32),
            pltpu.VMEM((DEG_R, 128), jnp.float32),
            pltpu.VMEM((DEG_R,), jnp.int32),
            pltpu.VMEM_SHARED((DEG_R, 128), jnp.float32),
        ],
    )
    return kfn(dst2, zeros_n, iota_r)


def _agg_pipeline(src_t, dst_t, nblk, hs_hbm, si_all, di_all, rows, sems,
                  acc_sh):
    pltpu.sync_copy(src_t, si_all.at[pl.ds(0, nblk)])
    pltpu.sync_copy(dst_t, di_all.at[pl.ds(0, nblk)])
    pltpu.async_copy(hs_hbm.at[si_all.at[0]], rows[0], sems[0])

    def body(g, carry):
        for b in range(2):
            j = 2 * g + b
            nxt = b ^ 1

            @pl.when(j + 1 < nblk)
            def _issue():
                pltpu.async_copy(hs_hbm.at[si_all.at[j + 1]], rows[nxt],
                                 sems[nxt])

            pltpu.make_async_copy(hs_hbm.at[si_all.at[j]], rows[b],
                                  sems[b]).wait()
            pltpu.sync_copy(rows[b], acc_sh.at[di_all.at[j]], add=True)
        return carry

    lax.fori_loop(0, nblk // 2, body, 0)


def _agg_body(hs_hbm, srcA, dstA, srcB, dstB, zeros_hbm, out_hbm,
              si_all, di_all, rows_a, rows_b, acc_sh, hs_sh, sem_a, sem_b):
    c = lax.axis_index("c")
    s = lax.axis_index("s")
    pltpu.sync_copy(zeros_hbm, acc_sh.at[pl.ds(s * STRIPE, STRIPE)])
    pltpu.sync_copy(hs_hbm.at[pl.ds(s * STRIPE, STRIPE)],
                    hs_sh.at[pl.ds(s * STRIPE, STRIPE)])
    plsc.subcore_barrier()

    rows = (rows_a, rows_b)
    sems = (sem_a, sem_b)

    @pl.when(c == 0)
    def _core0():
        _agg_pipeline(srcA.at[s], dstA.at[s], NBLK_A, hs_sh, si_all, di_all,
                      rows, sems, acc_sh)

    @pl.when(c == 1)
    def _core1():
        _agg_pipeline(srcB.at[s], dstB.at[s], NBLK_B, hs_sh, si_all, di_all,
                      rows, sems, acc_sh)

    plsc.subcore_barrier()
    pltpu.sync_copy(acc_sh.at[pl.ds(s * STRIPE, STRIPE)],
                    out_hbm.at[c].at[pl.ds(s * STRIPE, STRIPE)])


def _sc_aggregate(hs, srcA, dstA, srcB, dstB, zeros_half):
    kfn = pl.kernel(
        _agg_body,
        out_type=jax.ShapeDtypeStruct((NC, N_PAD, HP), jnp.float32),
        mesh=_mesh(),
        compiler_params=pltpu.CompilerParams(use_tc_tiling_on_sc=False),
        scratch_types=[
            pltpu.VMEM((NBLK_MAX, K2), jnp.int32),
            pltpu.VMEM((NBLK_MAX, K2), jnp.int32),
            pltpu.VMEM((K2, HP), jnp.float32),
            pltpu.VMEM((K2, HP), jnp.float32),
            pltpu.VMEM_SHARED((N_PAD, HP), jnp.float32),
            pltpu.VMEM_SHARED((N_PAD, HP), jnp.float32),
            pltpu.SemaphoreType.DMA,
            pltpu.SemaphoreType.DMA,
        ],
    )
    return kfn(hs, srcA, dstA, srcB, dstB, zeros_half)


def _tc1_body(x_ref, w_ref, deg_ref, hs_ref, dinv_ref):
    deg = deg_ref[0] + deg_ref[1] + 1.0
    dinv = lax.rsqrt(deg)[:, None]
    h = jnp.dot(x_ref[...], w_ref[...], preferred_element_type=jnp.float32)
    hs_ref[...] = h * dinv
    dinv_ref[...] = dinv


def _tc1(x, W1p, degp):
    return pl.pallas_call(
        _tc1_body,
        grid=(TGRID,),
        in_specs=[
            pl.BlockSpec((RB, D), lambda i: (i, 0)),
            pl.BlockSpec((D, HP), lambda i: (0, 0)),
            pl.BlockSpec((NC, RB), lambda i: (0, i)),
        ],
        out_specs=[
            pl.BlockSpec((RB, HP), lambda i: (i, 0)),
            pl.BlockSpec((RB, 1), lambda i: (i, 0)),
        ],
        out_shape=[
            jax.ShapeDtypeStruct((N_PAD, HP), jnp.float32),
            jax.ShapeDtypeStruct((N_PAD, 1), jnp.float32),
        ],
    )(x, W1p, degp)


def _tc2_body(acc_ref, hs_ref, dinv_ref, b_ref, w_ref, hs2_ref):
    dinv = dinv_ref[...]
    pre = dinv * (acc_ref[0] + acc_ref[1] + hs_ref[...]) + b_ref[...]
    out1 = jnp.maximum(pre, 0.0)
    h2 = jnp.dot(out1, w_ref[...], preferred_element_type=jnp.float32)
    hs2_ref[...] = h2 * dinv


def _tc2(acc1, hs1, dinv, b1p, W2p):
    return pl.pallas_call(
        _tc2_body,
        grid=(TGRID,),
        in_specs=[
            pl.BlockSpec((NC, RB, HP), lambda i: (0, i, 0)),
            pl.BlockSpec((RB, HP), lambda i: (i, 0)),
            pl.BlockSpec((RB, 1), lambda i: (i, 0)),
            pl.BlockSpec((1, HP), lambda i: (0, 0)),
            pl.BlockSpec((HP, HP), lambda i: (0, 0)),
        ],
        out_specs=pl.BlockSpec((RB, HP), lambda i: (i, 0)),
        out_shape=jax.ShapeDtypeStruct((N_PAD, HP), jnp.float32),
    )(acc1, hs1, dinv, b1p, W2p)


def _tc3_body(acc_ref, hs_ref, dinv_ref, b_ref, batch_ref, wfc_ref, bfc_ref,
              out_ref, pooled_acc, cnt_acc):
    i = pl.program_id(0)

    @pl.when(i == 0)
    def _init():
        pooled_acc[...] = jnp.zeros((G, HP), jnp.float32)
        cnt_acc[...] = jnp.zeros((G, 1), jnp.float32)

    pre = dinv_ref[...] * (acc_ref[0] + acc_ref[1] + hs_ref[...]) + b_ref[...]
    out2 = jnp.maximum(pre, 0.0)
    rid = i * RB + lax.broadcasted_iota(jnp.int32, (RB, 1), 0)
    valid = rid < N
    out2 = jnp.where(valid, out2, 0.0)
    oh = (batch_ref[...] ==
          lax.broadcasted_iota(jnp.int32, (1, G), 1)).astype(jnp.float32)
    oh = jnp.where(valid, oh, 0.0)
    dn = (((0,), (0,)), ((), ()))
    pooled_acc[...] += lax.dot_general(oh, out2, dn,
                                       preferred_element_type=jnp.float32)
    cnt_acc[...] += lax.dot_general(oh, jnp.ones((RB, 1), jnp.float32), dn,
                                    preferred_element_type=jnp.float32)

    @pl.when(i == TGRID - 1)
    def _fin():
        cnt = jnp.clip(cnt_acc[...], 1.0, None)
        pooled = pooled_acc[...] / cnt
        logits = (jnp.dot(pooled, wfc_ref[...], preferred_element_type=jnp.float32)
                  + bfc_ref[...])
        m = jnp.max(logits, axis=1, keepdims=True)
        z = logits - m
        lse = jnp.log(jnp.sum(jnp.exp(z), axis=1, keepdims=True))
        out_ref[...] = z - lse


def _tc3(acc2, hs2, dinv, b2p, batch2, Wfcp, bfc):
    return pl.pallas_call(
        _tc3_body,
        grid=(TGRID,),
        in_specs=[
            pl.BlockSpec((NC, RB, HP), lambda i: (0, i, 0)),
            pl.BlockSpec((RB, HP), lambda i: (i, 0)),
            pl.BlockSpec((RB, 1), lambda i: (i, 0)),
            pl.BlockSpec((1, HP), lambda i: (0, 0)),
            pl.BlockSpec((RB, 1), lambda i: (i, 0)),
            pl.BlockSpec((HP, C), lambda i: (0, 0)),
            pl.BlockSpec((1, C), lambda i: (0, 0)),
        ],
        out_specs=pl.BlockSpec((G, C), lambda i: (0, 0)),
        out_shape=jax.ShapeDtypeStruct((G, C), jnp.float32),
        scratch_shapes=[
            pltpu.VMEM((G, HP), jnp.float32),
            pltpu.VMEM((G, 1), jnp.float32),
        ],
    )(acc2, hs2, dinv, b2p, batch2, Wfcp, bfc)


@jax.jit
def _run(x, edge_index, batch, W1, b1, W2, b2, Wfc, bfc):
    src = edge_index[0]
    dst = edge_index[1]
    pad = jnp.full((E_PAD - E,), N, jnp.int32)
    src_p = jnp.concatenate([src, pad])
    dst_p = jnp.concatenate([dst, pad])
    ea = NS * NBLK_A * K2
    srcA = src_p[:ea].reshape(NS, NBLK_A, K2)
    srcB = src_p[ea:].reshape(NS, NBLK_B, K2)
    dstA = dst_p[:ea].reshape(NS, NBLK_A, K2)
    dstB = dst_p[ea:].reshape(NS, NBLK_B, K2)
    dst2 = dst_p.reshape(NW, EPT)

    zeros_n = jnp.zeros((DEG_R, 128), jnp.float32)
    iota_r = jnp.arange(DEG_R, dtype=jnp.int32)
    zeros_half = jnp.zeros((STRIPE, HP), jnp.float32)

    W1p = W1
    W2p = W2
    Wfcp = Wfc
    b1p = b1.reshape(1, HP)
    b2p = b2.reshape(1, HP)

    degp = _sc_degree(dst2, zeros_n, iota_r).reshape(NC, N_PAD)
    hs1, dinv = _tc1(x, W1p, degp)
    acc1 = _sc_aggregate(hs1, srcA, dstA, srcB, dstB, zeros_half)
    hs2 = _tc2(acc1, hs1, dinv, b1p, W2p)
    acc2 = _sc_aggregate(hs2, srcA, dstA, srcB, dstB, zeros_half)
    return _tc3(acc2, hs2, dinv, b2p, batch.reshape(N, 1),
                Wfcp, bfc.reshape(1, C))


def kernel(x, edge_index, batch, W1, b1, W2, b2, Wfc, bfc):
    return _run(x, edge_index, batch, W1, b1, W2, b2, Wfc, bfc)


# speedup vs baseline: 1.0074x; 1.0074x over previous
import jax
import jax.numpy as jnp
from jax import lax
from jax.experimental import pallas as pl
from jax.experimental.pallas import tpu as pltpu
from jax.experimental.pallas import tpu_sc as plsc

N = 10000
E = 320000
D = 128
H = 64
C = 10
G = 128

NC = 2
NS = 16
NW = NC * NS

K = 128
NBLK = 80
K2 = 64
NBLK2 = 160
NBLK_A = 160
NBLK_B = 160
NBLK_MAX = max(NBLK_A, NBLK_B)
EPT = K * NBLK
E_PAD = EPT * NW
N_PAD = 10240
STRIPE = N_PAD // NS
HALF = STRIPE // 2
HP = 64

RB = 1280
TGRID = N_PAD // RB

def _mesh():
    return plsc.VectorSubcoreMesh(core_axis_name="c", subcore_axis_name="s",
                                  num_cores=NC, num_subcores=NS)


DEG_R = N_PAD // 128


def _deg_body(dst2, zeros_n, iota_r, out_hbm, di_flat, table_v, idx_r, deg_sh):
    c = lax.axis_index("c")
    s = lax.axis_index("s")
    eid = c * NS + s
    pltpu.sync_copy(zeros_n, table_v)

    @pl.when(s == 0)
    def _zero_shared():
        pltpu.sync_copy(table_v, deg_sh)

    pltpu.sync_copy(dst2.at[eid], di_flat)
    pltpu.sync_copy(iota_r, idx_r)
    plsc.subcore_barrier()

    ones16 = jnp.ones((16,), jnp.float32)

    def body(i, carry):
        idx = di_flat[pl.ds(i * 16, 16)]
        plsc.addupdate_scatter(table_v, [idx >> 7, idx & 127], ones16)
        return carry

    lax.fori_loop(0, EPT // 16, body, 0)
    pltpu.sync_copy(table_v, deg_sh.at[idx_r], add=True)
    plsc.subcore_barrier()
    @pl.when(s < DEG_R // 8)
    def _copy_out():
        pltpu.sync_copy(deg_sh.at[pl.ds(s * 8, 8)],
                        table_v.at[pl.ds(0, 8)])
        pltpu.sync_copy(table_v.at[pl.ds(0, 8)],
                        out_hbm.at[c].at[pl.ds(s * 8, 8)])


def _sc_degree(dst2, zeros_n, iota_r):
    kfn = pl.kernel(
        _deg_body,
        out_type=jax.ShapeDtypeStruct((NC, DEG_R, 128), jnp.float32),
        mesh=_mesh(),
        compiler_params=pltpu.CompilerParams(needs_layout_passes=False),
        scratch_types=[
            pltpu.VMEM((EPT,), jnp.int32),
            pltpu.VMEM((DEG_R, 128), jnp.float32),
            pltpu.VMEM((DEG_R,), jnp.int32),
            pltpu.VMEM_SHARED((DEG_R, 128), jnp.float32),
        ],
    )
    return kfn(dst2, zeros_n, iota_r)


def _agg_pipeline(src_t, dst_t, nblk, hs_hbm, si_all, di_all, rows, sems,
                  acc_sh):
    pltpu.sync_copy(src_t, si_all.at[pl.ds(0, nblk)])
    pltpu.sync_copy(dst_t, di_all.at[pl.ds(0, nblk)])
    pltpu.async_copy(hs_hbm.at[si_all.at[0]], rows[0], sems[0])

    def body(g, carry):
        for b in range(2):
            j = 2 * g + b
            nxt = b ^ 1

            @pl.when(j + 1 < nblk)
            def _issue():
                pltpu.async_copy(hs_hbm.at[si_all.at[j + 1]], rows[nxt],
                                 sems[nxt])

            pltpu.make_async_copy(hs_hbm.at[si_all.at[j]], rows[b],
                                  sems[b]).wait()
            pltpu.sync_copy(rows[b], acc_sh.at[di_all.at[j]], add=True)
        return carry

    lax.fori_loop(0, nblk // 2, body, 0)


def _agg_body(hs_hbm, srcA, dstA, srcB, dstB, zeros_hbm, out_hbm,
              si_all, di_all, rows_a, rows_b, acc_sh, hs_sh, sem_a, sem_b):
    c = lax.axis_index("c")
    s = lax.axis_index("s")
    pltpu.sync_copy(zeros_hbm, acc_sh.at[pl.ds(s * STRIPE, STRIPE)])
    pltpu.sync_copy(hs_hbm.at[pl.ds(s * STRIPE, STRIPE)],
                    hs_sh.at[pl.ds(s * STRIPE, STRIPE)])
    plsc.subcore_barrier()

    rows = (rows_a, rows_b)
    sems = (sem_a, sem_b)

    @pl.when(c == 0)
    def _core0():
        _agg_pipeline(srcA.at[s], dstA.at[s], NBLK_A, hs_sh, si_all, di_all,
                      rows, sems, acc_sh)

    @pl.when(c == 1)
    def _core1():
        _agg_pipeline(srcB.at[s], dstB.at[s], NBLK_B, hs_sh, si_all, di_all,
                      rows, sems, acc_sh)

    plsc.subcore_barrier()
    pltpu.sync_copy(acc_sh.at[pl.ds(s * STRIPE, STRIPE)],
                    out_hbm.at[c].at[pl.ds(s * STRIPE, STRIPE)])


def _sc_aggregate(hs, srcA, dstA, srcB, dstB, zeros_half):
    kfn = pl.kernel(
        _agg_body,
        out_type=jax.ShapeDtypeStruct((NC, N_PAD, HP), jnp.float32),
        mesh=_mesh(),
        compiler_params=pltpu.CompilerParams(use_tc_tiling_on_sc=False),
        scratch_types=[
            pltpu.VMEM((NBLK_MAX, K2), jnp.int32),
            pltpu.VMEM((NBLK_MAX, K2), jnp.int32),
            pltpu.VMEM((K2, HP), jnp.float32),
            pltpu.VMEM((K2, HP), jnp.float32),
            pltpu.VMEM_SHARED((N_PAD, HP), jnp.float32),
            pltpu.VMEM_SHARED((N_PAD, HP), jnp.float32),
            pltpu.SemaphoreType.DMA,
            pltpu.SemaphoreType.DMA,
        ],
    )
    return kfn(hs, srcA, dstA, srcB, dstB, zeros_half)


def _tc1_body(x_ref, w_ref, deg_ref, hs_ref, dinv_ref):
    deg = deg_ref[0] + deg_ref[1] + 1.0
    dinv = lax.rsqrt(deg)[:, None]
    h = jnp.dot(x_ref[...], w_ref[...], preferred_element_type=jnp.float32)
    hs_ref[...] = h * dinv
    dinv_ref[...] = dinv


def _tc1(x, W1p, degp):
    return pl.pallas_call(
        _tc1_body,
        grid=(TGRID,),
        in_specs=[
            pl.BlockSpec((RB, D), lambda i: (i, 0)),
            pl.BlockSpec((D, HP), lambda i: (0, 0)),
            pl.BlockSpec((NC, RB), lambda i: (0, i)),
        ],
        out_specs=[
            pl.BlockSpec((RB, HP), lambda i: (i, 0)),
            pl.BlockSpec((RB, 1), lambda i: (i, 0)),
        ],
        out_shape=[
            jax.ShapeDtypeStruct((N_PAD, HP), jnp.float32),
            jax.ShapeDtypeStruct((N_PAD, 1), jnp.float32),
        ],
    )(x, W1p, degp)


def _tc2_body(acc_ref, hs_ref, dinv_ref, b_ref, w_ref, hs2_ref):
    dinv = dinv_ref[...]
    pre = dinv * (acc_ref[0] + acc_ref[1] + hs_ref[...]) + b_ref[...]
    out1 = jnp.maximum(pre, 0.0)
    h2 = jnp.dot(out1, w_ref[...], preferred_element_type=jnp.float32)
    hs2_ref[...] = h2 * dinv


def _tc2(acc1, hs1, dinv, b1p, W2p):
    return pl.pallas_call(
        _tc2_body,
        grid=(TGRID,),
        in_specs=[
            pl.BlockSpec((NC, RB, HP), lambda i: (0, i, 0)),
            pl.BlockSpec((RB, HP), lambda i: (i, 0)),
            pl.BlockSpec((RB, 1), lambda i: (i, 0)),
            pl.BlockSpec((1, HP), lambda i: (0, 0)),
            pl.BlockSpec((HP, HP), lambda i: (0, 0)),
        ],
        out_specs=pl.BlockSpec((RB, HP), lambda i: (i, 0)),
        out_shape=jax.ShapeDtypeStruct((N_PAD, HP), jnp.float32),
    )(acc1, hs1, dinv, b1p, W2p)


def _tc3_body(acc_ref, hs_ref, dinv_ref, b_ref, batch_ref, wfc_ref, bfc_ref,
              out_ref, pooled_acc, cnt_acc):
    i = pl.program_id(0)

    @pl.when(i == 0)
    def _init():
        pooled_acc[...] = jnp.zeros((G, HP), jnp.float32)
        cnt_acc[...] = jnp.zeros((G, 1), jnp.float32)

    pre = dinv_ref[...] * (acc_ref[0] + acc_ref[1] + hs_ref[...]) + b_ref[...]
    out2 = jnp.maximum(pre, 0.0)
    rid = i * RB + lax.broadcasted_iota(jnp.int32, (RB, 1), 0)
    valid = rid < N
    out2 = jnp.where(valid, out2, 0.0)
    oh = (batch_ref[...] ==
          lax.broadcasted_iota(jnp.int32, (1, G), 1)).astype(jnp.float32)
    oh = jnp.where(valid, oh, 0.0)
    dn = (((0,), (0,)), ((), ()))
    pooled_acc[...] += lax.dot_general(oh, out2, dn,
                                       preferred_element_type=jnp.float32)
    cnt_acc[...] += lax.dot_general(oh, jnp.ones((RB, 1), jnp.float32), dn,
                                    preferred_element_type=jnp.float32)

    @pl.when(i == TGRID - 1)
    def _fin():
        cnt = jnp.clip(cnt_acc[...], 1.0, None)
        pooled = pooled_acc[...] / cnt
        logits = (jnp.dot(pooled, wfc_ref[...], preferred_element_type=jnp.float32)
                  + bfc_ref[...])
        m = jnp.max(logits, axis=1, keepdims=True)
        z = logits - m
        lse = jnp.log(jnp.sum(jnp.exp(z), axis=1, keepdims=True))
        out_ref[...] = z - lse


def _tc3(acc2, hs2, dinv, b2p, batch2, Wfcp, bfc):
    return pl.pallas_call(
        _tc3_body,
        grid=(TGRID,),
        in_specs=[
            pl.BlockSpec((NC, RB, HP), lambda i: (0, i, 0)),
            pl.BlockSpec((RB, HP), lambda i: (i, 0)),
            pl.BlockSpec((RB, 1), lambda i: (i, 0)),
            pl.BlockSpec((1, HP), lambda i: (0, 0)),
            pl.BlockSpec((RB, 1), lambda i: (i, 0)),
            pl.BlockSpec((HP, C), lambda i: (0, 0)),
            pl.BlockSpec((1, C), lambda i: (0, 0)),
        ],
        out_specs=pl.BlockSpec((G, C), lambda i: (0, 0)),
        out_shape=jax.ShapeDtypeStruct((G, C), jnp.float32),
        scratch_shapes=[
            pltpu.VMEM((G, HP), jnp.float32),
            pltpu.VMEM((G, 1), jnp.float32),
        ],
    )(acc2, hs2, dinv, b2p, batch2, Wfcp, bfc)


@jax.jit
def _run(x, edge_index, batch, W1, b1, W2, b2, Wfc, bfc):
    src = edge_index[0]
    dst = edge_index[1]
    pad = jnp.full((E_PAD - E,), N, jnp.int32)
    src_p = jnp.concatenate([src, pad])
    dst_p = jnp.concatenate([dst, pad])
    ea = NS * NBLK_A * K2
    srcA = src_p[:ea].reshape(NS, NBLK_A, K2)
    srcB = src_p[ea:].reshape(NS, NBLK_B, K2)
    dstA = dst_p[:ea].reshape(NS, NBLK_A, K2)
    dstB = dst_p[ea:].reshape(NS, NBLK_B, K2)
    dst2 = dst_p.reshape(NW, EPT)

    zeros_n = jnp.zeros((DEG_R, 128), jnp.float32)
    iota_r = jnp.arange(DEG_R, dtype=jnp.int32)
    zeros_half = jnp.zeros((STRIPE, HP), jnp.float32)

    W1p = W1
    W2p = W2
    Wfcp = Wfc
    b1p = b1.reshape(1, HP)
    b2p = b2.reshape(1, HP)

    degp = _sc_degree(dst2, zeros_n, iota_r).reshape(NC, N_PAD)
    hs1, dinv = _tc1(x, W1p, degp)
    acc1 = _sc_aggregate(hs1, srcA, dstA, srcB, dstB, zeros_half)
    hs2 = _tc2(acc1, hs1, dinv, b1p, W2p)
    acc2 = _sc_aggregate(hs2, srcA, dstA, srcB, dstB, zeros_half)
    return _tc3(acc2, hs2, dinv, b2p, batch.reshape(N, 1),
                Wfcp, bfc.reshape(1, C))


def kernel(x, edge_index, batch, W1, b1, W2, b2, Wfc, bfc):
    return _run(x, edge_index, batch, W1, b1, W2, b2, Wfc, bfc)
